# baseline (device time: 23149 ns/iter reference)
def kernel(x, A, B, C):
    import jax
    import jax.numpy as jnp
    from jax import lax
    from jax.experimental import pallas as pl
    from jax.experimental.pallas import tpu as pltpu

    Bdim, S, D = x.shape
    N = A.shape[1]
    TB = 8

    A_t = A.T

    def body(x_ref, a_ref, b_ref, c_ref, out_ref, hp_ref, send_sem, recv_sem):
        my_x = lax.axis_index("x")
        my_y = lax.axis_index("y")

        dA = jnp.exp(a_ref[...])[None]

        @pl.when(my_x == 1)
        def _():
            recv = pltpu.make_async_remote_copy(
                src_ref=hp_ref,
                dst_ref=hp_ref,
                send_sem=send_sem,
                recv_sem=recv_sem,
                device_id=(0, my_y),
                device_id_type=pl.DeviceIdType.MESH,
            )
            recv.wait_recv()

        zero = jnp.zeros((Bdim, N, D), jnp.float32)
        h0 = jnp.where(my_x == 0, zero, hp_ref[...])

        nblk = S // TB

        def blk(i, h):
            t0 = pl.multiple_of(i * TB, TB)
            xblk = x_ref[:, pl.ds(t0, TB), :]
            bblk = b_ref[:, pl.ds(t0, TB), :]
            cblk = c_ref[:, pl.ds(t0, TB), :]
            ys = []
            for j in range(TB):
                xt = xblk[:, j, :]
                bt = bblk[:, j, :]
                ct = cblk[:, j, :]
                h = h * dA + xt[:, None, :] * bt[:, :, None]
                ys.append(jnp.sum(h * ct[:, :, None], axis=1))
            out_ref[:, pl.ds(t0, TB), :] = jnp.stack(ys, axis=1)
            return h

        h_final = lax.fori_loop(0, nblk, blk, h0)

        @pl.when(my_x == 0)
        def _():
            hp_ref[...] = h_final
            send = pltpu.make_async_remote_copy(
                src_ref=hp_ref,
                dst_ref=hp_ref,
                send_sem=send_sem,
                recv_sem=recv_sem,
                device_id=(1, my_y),
                device_id_type=pl.DeviceIdType.MESH,
            )
            send.start()
            send.wait_send()

    return pl.pallas_call(
        body,
        out_shape=jax.ShapeDtypeStruct((Bdim, S, D), jnp.float32),
        in_specs=[pl.BlockSpec(memory_space=pltpu.VMEM)] * 4,
        out_specs=pl.BlockSpec(memory_space=pltpu.VMEM),
        scratch_shapes=[
            pltpu.VMEM((Bdim, N, D), jnp.float32),
            pltpu.SemaphoreType.DMA,
            pltpu.SemaphoreType.DMA,
        ],
    )(x, A_t, B, C)
